# initial kernel scaffold (unmeasured)
import jax
import jax.numpy as jnp
from jax import lax
from jax.experimental import pallas as pl
from jax.experimental.pallas import tpu as pltpu

N_CHUNKS = 8


def kernel(A, B):
    M, K = A.shape
    _, N = B.shape
    bm = M // N_CHUNKS

    A = A.astype(jnp.bfloat16)
    B = B.astype(jnp.bfloat16)

    def body(a_ref, b_ref, out_ref, send_buf, recv_buf, send_sems, recv_sems):
        i = pl.program_id(0)
        my_x = lax.axis_index("x")
        my_y = lax.axis_index("y")
        peer = (my_x, 1 - my_y)

        @pl.when(i == 0)
        def _():
            barrier = pltpu.get_barrier_semaphore()
            pl.semaphore_signal(
                barrier, inc=1, device_id=peer,
                device_id_type=pl.DeviceIdType.MESH,
            )
            pl.semaphore_wait(barrier, 1)

        slot = i % 2
        partial = jnp.dot(
            a_ref[...], b_ref[...], preferred_element_type=jnp.float32
        )
        send_buf[slot] = partial.astype(jnp.bfloat16)

        rdma = pltpu.make_async_remote_copy(
            src_ref=send_buf.at[slot],
            dst_ref=recv_buf.at[slot],
            send_sem=send_sems.at[slot],
            recv_sem=recv_sems.at[slot],
            device_id=peer,
            device_id_type=pl.DeviceIdType.MESH,
        )
        rdma.start()
        rdma.wait()

        out_ref[...] = send_buf[slot] + recv_buf[slot]

    return pl.pallas_call(
        body,
        grid=(N_CHUNKS,),
        in_specs=[
            pl.BlockSpec((bm, K), lambda i: (i, 0)),
            pl.BlockSpec((K, N), lambda i: (0, 0)),
        ],
        out_specs=pl.BlockSpec((bm, N), lambda i: (i, 0)),
        out_shape=jax.ShapeDtypeStruct((M, N), jnp.bfloat16),
        scratch_shapes=[
            pltpu.VMEM((2, bm, N), jnp.bfloat16),
            pltpu.VMEM((2, bm, N), jnp.bfloat16),
            pltpu.SemaphoreType.DMA((2,)),
            pltpu.SemaphoreType.DMA((2,)),
        ],
        compiler_params=pltpu.CompilerParams(
            dimension_semantics=("arbitrary",),
            collective_id=0,
        ),
    )(A, B)


# baseline (device time: 519456 ns/iter reference)
import jax
import jax.numpy as jnp
from jax import lax
from jax.experimental import pallas as pl
from jax.experimental.pallas import tpu as pltpu

N_CHUNKS = 8


def kernel(A, B):
    M, K = A.shape
    _, N = B.shape
    bm = M // N_CHUNKS

    A = A.astype(jnp.bfloat16)
    B = B.astype(jnp.bfloat16)

    def body(a_ref, b_ref, out_ref, send_buf, recv_buf, send_sems, recv_sems):
        i = pl.program_id(0)
        my_x = lax.axis_index("x")
        my_y = lax.axis_index("y")
        peer = (my_x, 1 - my_y)

        @pl.when(i == 0)
        def _():
            barrier = pltpu.get_barrier_semaphore()
            pl.semaphore_signal(
                barrier, inc=1, device_id=peer,
                device_id_type=pl.DeviceIdType.MESH,
            )
            pl.semaphore_wait(barrier, 1)

        slot = i % 2
        partial = jnp.dot(
            a_ref[...], b_ref[...], preferred_element_type=jnp.float32
        )
        send_buf[slot] = partial.astype(jnp.bfloat16)

        rdma = pltpu.make_async_remote_copy(
            src_ref=send_buf.at[slot],
            dst_ref=recv_buf.at[slot],
            send_sem=send_sems.at[slot],
            recv_sem=recv_sems.at[slot],
            device_id=peer,
            device_id_type=pl.DeviceIdType.MESH,
        )
        rdma.start()
        rdma.wait()

        out_ref[...] = send_buf[slot] + recv_buf[slot]

    return pl.pallas_call(
        body,
        grid=(N_CHUNKS,),
        in_specs=[
            pl.BlockSpec((bm, K), lambda i: (i, 0)),
            pl.BlockSpec((K, N), lambda i: (0, 0)),
        ],
        out_specs=pl.BlockSpec((bm, N), lambda i: (i, 0)),
        out_shape=jax.ShapeDtypeStruct((M, N), jnp.bfloat16),
        scratch_shapes=[
            pltpu.VMEM((2, bm, N), jnp.bfloat16),
            pltpu.VMEM((2, bm, N), jnp.bfloat16),
            pltpu.SemaphoreType.DMA((2,)),
            pltpu.SemaphoreType.DMA((2,)),
        ],
        compiler_params=pltpu.CompilerParams(
            dimension_semantics=("arbitrary",),
            collective_id=0,
            vmem_limit_bytes=56 * 1024 * 1024,
        ),
    )(A, B)


# device time: 449025 ns/iter; 1.1569x vs baseline; 1.1569x over previous
import jax
import jax.numpy as jnp
from jax import lax
from jax.experimental import pallas as pl
from jax.experimental.pallas import tpu as pltpu

N_CHUNKS = 8


def kernel(A, B):
    M, K = A.shape
    _, N = B.shape
    bm = M // N_CHUNKS

    A = A.astype(jnp.bfloat16)
    B = B.astype(jnp.bfloat16)

    def make_rdma(j, send_buf, recv_buf, send_sems, recv_sems, peer):
        return pltpu.make_async_remote_copy(
            src_ref=send_buf.at[j % 2],
            dst_ref=recv_buf.at[j % 4],
            send_sem=send_sems.at[j % 2],
            recv_sem=recv_sems.at[j % 4],
            device_id=peer,
            device_id_type=pl.DeviceIdType.MESH,
        )

    def body(a_ref, b_ref, out_ref, send_buf, recv_buf, send_sems, recv_sems):
        i = pl.program_id(0)
        my_x = lax.axis_index("x")
        my_y = lax.axis_index("y")
        peer = (my_x, 1 - my_y)

        @pl.when(i == 0)
        def _():
            barrier = pltpu.get_barrier_semaphore()
            pl.semaphore_signal(
                barrier, inc=1, device_id=peer,
                device_id_type=pl.DeviceIdType.MESH,
            )
            pl.semaphore_wait(barrier, 1)

        @pl.when(i < N_CHUNKS)
        def _():
            partial = jnp.dot(
                a_ref[...], b_ref[...], preferred_element_type=jnp.float32
            )
            send_buf[i % 2] = partial.astype(jnp.bfloat16)

        @pl.when(i > 0)
        def _():
            prev = i - 1
            make_rdma(prev, send_buf, recv_buf, send_sems, recv_sems,
                      peer).wait()

        @pl.when(i < N_CHUNKS)
        def _():
            make_rdma(i, send_buf, recv_buf, send_sems, recv_sems,
                      peer).start()

        @pl.when(i > 0)
        def _():
            prev = i - 1
            out_ref[...] = send_buf[prev % 2] + recv_buf[prev % 4]

    return pl.pallas_call(
        body,
        grid=(N_CHUNKS + 1,),
        in_specs=[
            pl.BlockSpec((bm, K), lambda i: (jnp.minimum(i, N_CHUNKS - 1), 0)),
            pl.BlockSpec((K, N), lambda i: (0, 0)),
        ],
        out_specs=pl.BlockSpec((bm, N), lambda i: (jnp.maximum(i - 1, 0), 0)),
        out_shape=jax.ShapeDtypeStruct((M, N), jnp.bfloat16),
        scratch_shapes=[
            pltpu.VMEM((2, bm, N), jnp.bfloat16),
            pltpu.VMEM((4, bm, N), jnp.bfloat16),
            pltpu.SemaphoreType.DMA((2,)),
            pltpu.SemaphoreType.DMA((4,)),
        ],
        compiler_params=pltpu.CompilerParams(
            dimension_semantics=("arbitrary",),
            collective_id=0,
            vmem_limit_bytes=56 * 1024 * 1024,
        ),
    )(A, B)


# device time: 415093 ns/iter; 1.2514x vs baseline; 1.0817x over previous
import jax
import jax.numpy as jnp
from jax import lax
from jax.experimental import pallas as pl
from jax.experimental.pallas import tpu as pltpu

N_CHUNKS = 16
N_SEND = 3
N_RECV = 5


def kernel(A, B):
    M, K = A.shape
    _, N = B.shape
    bm = M // N_CHUNKS

    B = B.astype(jnp.bfloat16)

    def make_rdma(j, send_buf, recv_buf, send_sems, recv_sems, peer):
        return pltpu.make_async_remote_copy(
            src_ref=send_buf.at[j % N_SEND],
            dst_ref=recv_buf.at[j % N_RECV],
            send_sem=send_sems.at[j % N_SEND],
            recv_sem=recv_sems.at[j % N_RECV],
            device_id=peer,
            device_id_type=pl.DeviceIdType.MESH,
        )

    def body(a_ref, b_ref, out_ref, send_buf, recv_buf, send_sems, recv_sems):
        i = pl.program_id(0)
        my_x = lax.axis_index("x")
        my_y = lax.axis_index("y")
        peer = (my_x, 1 - my_y)

        @pl.when(i == 0)
        def _():
            barrier = pltpu.get_barrier_semaphore()
            pl.semaphore_signal(
                barrier, inc=1, device_id=peer,
                device_id_type=pl.DeviceIdType.MESH,
            )
            pl.semaphore_wait(barrier, 1)

        @pl.when(i >= 2)
        def _():
            make_rdma(i - 2, send_buf, recv_buf, send_sems, recv_sems,
                      peer).wait()

        @pl.when(i < N_CHUNKS)
        def _():
            partial = jnp.dot(
                a_ref[...].astype(jnp.bfloat16),
                b_ref[...],
                preferred_element_type=jnp.float32,
            )
            send_buf[i % N_SEND] = partial.astype(jnp.bfloat16)
            make_rdma(i, send_buf, recv_buf, send_sems, recv_sems,
                      peer).start()

        @pl.when(i >= 2)
        def _():
            prev = i - 2
            out_ref[...] = send_buf[prev % N_SEND] + recv_buf[prev % N_RECV]

    return pl.pallas_call(
        body,
        grid=(N_CHUNKS + 2,),
        in_specs=[
            pl.BlockSpec((bm, K), lambda i: (jnp.minimum(i, N_CHUNKS - 1), 0)),
            pl.BlockSpec((K, N), lambda i: (0, 0)),
        ],
        out_specs=pl.BlockSpec((bm, N), lambda i: (jnp.maximum(i - 2, 0), 0)),
        out_shape=jax.ShapeDtypeStruct((M, N), jnp.bfloat16),
        scratch_shapes=[
            pltpu.VMEM((N_SEND, bm, N), jnp.bfloat16),
            pltpu.VMEM((N_RECV, bm, N), jnp.bfloat16),
            pltpu.SemaphoreType.DMA((N_SEND,)),
            pltpu.SemaphoreType.DMA((N_RECV,)),
        ],
        compiler_params=pltpu.CompilerParams(
            dimension_semantics=("arbitrary",),
            collective_id=0,
            vmem_limit_bytes=56 * 1024 * 1024,
        ),
    )(A, B)


# device time: 412240 ns/iter; 1.2601x vs baseline; 1.0069x over previous
import jax
import jax.numpy as jnp
from jax import lax
from jax.experimental import pallas as pl
from jax.experimental.pallas import tpu as pltpu

N_CHUNKS = 32
N_SEND = 3
N_RECV = 5


def kernel(A, B):
    M, K = A.shape
    _, N = B.shape
    bm = M // N_CHUNKS

    B = B.astype(jnp.bfloat16)

    def make_rdma(j, send_buf, recv_buf, send_sems, recv_sems, peer):
        return pltpu.make_async_remote_copy(
            src_ref=send_buf.at[j % N_SEND],
            dst_ref=recv_buf.at[j % N_RECV],
            send_sem=send_sems.at[j % N_SEND],
            recv_sem=recv_sems.at[j % N_RECV],
            device_id=peer,
            device_id_type=pl.DeviceIdType.MESH,
        )

    def body(a_ref, b_ref, out_ref, send_buf, recv_buf, send_sems, recv_sems):
        i = pl.program_id(0)
        my_x = lax.axis_index("x")
        my_y = lax.axis_index("y")
        peer = (my_x, 1 - my_y)

        @pl.when(i == 0)
        def _():
            barrier = pltpu.get_barrier_semaphore()
            pl.semaphore_signal(
                barrier, inc=1, device_id=peer,
                device_id_type=pl.DeviceIdType.MESH,
            )
            pl.semaphore_wait(barrier, 1)

        @pl.when(i >= 2)
        def _():
            make_rdma(i - 2, send_buf, recv_buf, send_sems, recv_sems,
                      peer).wait()

        @pl.when(i < N_CHUNKS)
        def _():
            partial = jnp.dot(
                a_ref[...].astype(jnp.bfloat16),
                b_ref[...],
                preferred_element_type=jnp.float32,
            )
            send_buf[i % N_SEND] = partial.astype(jnp.bfloat16)
            make_rdma(i, send_buf, recv_buf, send_sems, recv_sems,
                      peer).start()

        @pl.when(i >= 2)
        def _():
            prev = i - 2
            out_ref[...] = send_buf[prev % N_SEND] + recv_buf[prev % N_RECV]

    return pl.pallas_call(
        body,
        grid=(N_CHUNKS + 2,),
        in_specs=[
            pl.BlockSpec((bm, K), lambda i: (jnp.minimum(i, N_CHUNKS - 1), 0)),
            pl.BlockSpec((K, N), lambda i: (0, 0)),
        ],
        out_specs=pl.BlockSpec((bm, N), lambda i: (jnp.maximum(i - 2, 0), 0)),
        out_shape=jax.ShapeDtypeStruct((M, N), jnp.bfloat16),
        scratch_shapes=[
            pltpu.VMEM((N_SEND, bm, N), jnp.bfloat16),
            pltpu.VMEM((N_RECV, bm, N), jnp.bfloat16),
            pltpu.SemaphoreType.DMA((N_SEND,)),
            pltpu.SemaphoreType.DMA((N_RECV,)),
        ],
        compiler_params=pltpu.CompilerParams(
            dimension_semantics=("arbitrary",),
            collective_id=0,
            vmem_limit_bytes=56 * 1024 * 1024,
        ),
    )(A, B)
